# baseline (speedup 1.0000x reference)
import numpy as np

import jax
import jax.numpy as jnp
from jax.experimental import pallas as pl
from jax.experimental.pallas import tpu as pltpu
from jax.sharding import Mesh, NamedSharding, PartitionSpec as P

TEMPERATURE = 0.1
EPS_LN = 1e-5
EPS_NORM = 1e-12

BM = 256


def _router_kernel(x_ref, w1_ref, b1_ref, gamma_ref, beta_ref,
                   w2_ref, b2_ref, anchors_ref,
                   proj_ref, probs_ref, ids_ref, a_scratch):
    @pl.when(pl.program_id(0) == 0)
    def _():
        a = anchors_ref[...]
        an = jnp.sqrt(jnp.sum(a * a, axis=-1, keepdims=True))
        a_scratch[...] = (a / jnp.maximum(an, EPS_NORM)).astype(jnp.bfloat16)

    h = jnp.dot(x_ref[...].astype(jnp.bfloat16), w1_ref[...],
                preferred_element_type=jnp.float32)
    h = h + b1_ref[...]
    mu = jnp.mean(h, axis=-1, keepdims=True)
    var = jnp.mean((h - mu) ** 2, axis=-1, keepdims=True)
    h = (h - mu) / jnp.sqrt(var + EPS_LN) * gamma_ref[...] + beta_ref[...]
    h = jnp.maximum(h, 0.0)

    p = jnp.dot(h.astype(jnp.bfloat16), w2_ref[...],
                preferred_element_type=jnp.float32)
    p = p + b2_ref[...]
    n = jnp.sqrt(jnp.sum(p * p, axis=-1, keepdims=True))
    projected = p / jnp.maximum(n, EPS_NORM)
    proj_ref[...] = projected
    n2 = jnp.sqrt(jnp.sum(projected * projected, axis=-1, keepdims=True))
    f = projected / jnp.maximum(n2, EPS_NORM)

    sims = jnp.dot(f.astype(jnp.bfloat16), a_scratch[...].T,
                   preferred_element_type=jnp.float32)
    logits = sims / TEMPERATURE
    m = jnp.max(logits, axis=-1, keepdims=True)
    e = jnp.exp(logits - m)
    probs = e / jnp.sum(e, axis=-1, keepdims=True)
    probs_ref[...] = probs
    ids_ref[...] = jnp.argmax(probs, axis=-1, keepdims=True).astype(jnp.int32)


def _router_shard(x, w1b, b1r, gammar, betar, w2b, b2r, anchors):
    b_, d_in = x.shape
    d_h = w1b.shape[1]
    d_a = w2b.shape[1]
    n_c = anchors.shape[0]
    grid = (b_ // BM,)

    return pl.pallas_call(
        _router_kernel,
        grid=grid,
        in_specs=[
            pl.BlockSpec((BM, d_in), lambda i: (i, 0)),
            pl.BlockSpec((d_in, d_h), lambda i: (0, 0)),
            pl.BlockSpec((1, d_h), lambda i: (0, 0)),
            pl.BlockSpec((1, d_h), lambda i: (0, 0)),
            pl.BlockSpec((1, d_h), lambda i: (0, 0)),
            pl.BlockSpec((d_h, d_a), lambda i: (0, 0)),
            pl.BlockSpec((1, d_a), lambda i: (0, 0)),
            pl.BlockSpec((n_c, d_a), lambda i: (0, 0)),
        ],
        out_specs=[
            pl.BlockSpec((BM, d_a), lambda i: (i, 0)),
            pl.BlockSpec((BM, n_c), lambda i: (i, 0)),
            pl.BlockSpec((BM, 1), lambda i: (i, 0)),
        ],
        out_shape=[
            jax.ShapeDtypeStruct((b_, d_a), jnp.float32),
            jax.ShapeDtypeStruct((b_, n_c), jnp.float32),
            jax.ShapeDtypeStruct((b_, 1), jnp.int32),
        ],
        scratch_shapes=[pltpu.VMEM((n_c, d_a), jnp.bfloat16)],
    )(x, w1b, b1r, gammar, betar, w2b, b2r, anchors)


@jax.jit
def kernel(x, W1, b1, gamma, beta, W2, b2, cluster_anchors):
    b_, d_in = x.shape
    d_h = W1.shape[1]
    d_a = W2.shape[1]

    devs = jax.devices()
    nd = 2 if (len(devs) >= 2 and b_ % (2 * BM) == 0) else 1
    mesh = Mesh(np.array(devs[:nd]), ("d",))
    row_sh = NamedSharding(mesh, P("d", None))
    repl = NamedSharding(mesh, P(None, None))

    xs = jax.device_put(x, row_sh)
    w1b = jax.device_put(W1.astype(jnp.bfloat16), repl)
    w2b = jax.device_put(W2.astype(jnp.bfloat16), repl)
    b1r = jax.device_put(b1.reshape(1, d_h), repl)
    gammar = jax.device_put(gamma.reshape(1, d_h), repl)
    betar = jax.device_put(beta.reshape(1, d_h), repl)
    b2r = jax.device_put(b2.reshape(1, d_a), repl)
    anchorsr = jax.device_put(cluster_anchors, repl)

    shard_fn = jax.shard_map(
        _router_shard,
        mesh=mesh,
        in_specs=(P("d", None),) + (P(None, None),) * 7,
        out_specs=(P("d", None), P("d", None), P("d", None)),
        check_vma=False,
    )
    projected, probs, ids = shard_fn(xs, w1b, b1r, gammar, betar, w2b, b2r,
                                     anchorsr)

    dev0 = devs[0]
    projected = jax.device_put(projected, dev0)
    probs = jax.device_put(probs, dev0)
    ids = jax.device_put(ids, dev0)
    return ids.reshape(b_), probs, projected


# speedup vs baseline: 1.7931x; 1.7931x over previous
import jax
import jax.numpy as jnp
from jax.experimental import pallas as pl
from jax.experimental.pallas import tpu as pltpu

TEMPERATURE = 0.1
EPS_LN = 1e-5
EPS_NORM = 1e-12

BM = 256


def _router_kernel(x_ref, w1_ref, b1_ref, gamma_ref, beta_ref,
                   w2_ref, b2_ref, anchors_ref,
                   proj_ref, probs_ref, ids_ref, a_scratch):
    @pl.when(pl.program_id(0) == 0)
    def _():
        a = anchors_ref[...]
        an = jnp.sqrt(jnp.sum(a * a, axis=-1, keepdims=True))
        a_scratch[...] = (a / jnp.maximum(an, EPS_NORM)).astype(jnp.bfloat16)

    h = jnp.dot(x_ref[...].astype(jnp.bfloat16), w1_ref[...],
                preferred_element_type=jnp.float32)
    h = h + b1_ref[...]
    mu = jnp.mean(h, axis=-1, keepdims=True)
    var = jnp.mean((h - mu) ** 2, axis=-1, keepdims=True)
    h = (h - mu) / jnp.sqrt(var + EPS_LN) * gamma_ref[...] + beta_ref[...]
    h = jnp.maximum(h, 0.0)

    p = jnp.dot(h.astype(jnp.bfloat16), w2_ref[...],
                preferred_element_type=jnp.float32)
    p = p + b2_ref[...]
    n = jnp.sqrt(jnp.sum(p * p, axis=-1, keepdims=True))
    projected = p / jnp.maximum(n, EPS_NORM)
    proj_ref[...] = projected
    n2 = jnp.sqrt(jnp.sum(projected * projected, axis=-1, keepdims=True))
    f = projected / jnp.maximum(n2, EPS_NORM)

    sims = jnp.dot(f.astype(jnp.bfloat16), a_scratch[...].T,
                   preferred_element_type=jnp.float32)
    logits = sims / TEMPERATURE
    m = jnp.max(logits, axis=-1, keepdims=True)
    e = jnp.exp(logits - m)
    probs = e / jnp.sum(e, axis=-1, keepdims=True)
    probs_ref[...] = probs
    ids_ref[...] = jnp.argmax(probs, axis=-1, keepdims=True).astype(jnp.int32)


@jax.jit
def kernel(x, W1, b1, gamma, beta, W2, b2, cluster_anchors):
    b_, d_in = x.shape
    d_h = W1.shape[1]
    d_a = W2.shape[1]
    n_c = cluster_anchors.shape[0]
    grid = (b_ // BM,)

    projected, probs, ids = pl.pallas_call(
        _router_kernel,
        grid=grid,
        in_specs=[
            pl.BlockSpec((BM, d_in), lambda i: (i, 0)),
            pl.BlockSpec((d_in, d_h), lambda i: (0, 0)),
            pl.BlockSpec((1, d_h), lambda i: (0, 0)),
            pl.BlockSpec((1, d_h), lambda i: (0, 0)),
            pl.BlockSpec((1, d_h), lambda i: (0, 0)),
            pl.BlockSpec((d_h, d_a), lambda i: (0, 0)),
            pl.BlockSpec((1, d_a), lambda i: (0, 0)),
            pl.BlockSpec((n_c, d_a), lambda i: (0, 0)),
        ],
        out_specs=[
            pl.BlockSpec((BM, d_a), lambda i: (i, 0)),
            pl.BlockSpec((BM, n_c), lambda i: (i, 0)),
            pl.BlockSpec((BM, 1), lambda i: (i, 0)),
        ],
        out_shape=[
            jax.ShapeDtypeStruct((b_, d_a), jnp.float32),
            jax.ShapeDtypeStruct((b_, n_c), jnp.float32),
            jax.ShapeDtypeStruct((b_, 1), jnp.int32),
        ],
        scratch_shapes=[pltpu.VMEM((n_c, d_a), jnp.bfloat16)],
    )(x, W1.astype(jnp.bfloat16), b1.reshape(1, d_h),
      gamma.reshape(1, d_h), beta.reshape(1, d_h), W2.astype(jnp.bfloat16),
      b2.reshape(1, d_a), cluster_anchors)

    return ids.reshape(b_), probs, projected
